# speedup vs baseline: 3.3382x; 3.3382x over previous
import functools

import jax
import jax.numpy as jnp
from jax import lax
from jax.experimental import pallas as pl
from jax.experimental.pallas import tpu as pltpu
from jax.experimental.pallas import tpu_sc as plsc

N_V = 100000
D_E = 128
B_TOTAL = 4096 * 50

_info = plsc.get_sparse_core_info()
_NC, _NS = _info.num_cores, _info.num_subcores
_NW = _NC * _NS
_BPW = B_TOTAL // _NW
_CHUNK = 128
_NBUF = 2
_NCHUNKS = _BPW // _CHUNK
_NGROUPS = _NCHUNKS // _NBUF


def _gather_fn():
    mesh = plsc.VectorSubcoreMesh(core_axis_name="c", subcore_axis_name="s")

    @functools.partial(
        pl.kernel,
        mesh=mesh,
        out_type=jax.ShapeDtypeStruct((B_TOTAL, D_E), jnp.float32),
        scratch_types=[
            pltpu.VMEM((_BPW,), jnp.int32),
            pltpu.VMEM((_NBUF, _CHUNK, D_E), jnp.float32),
            pltpu.SemaphoreType.DMA,
        ],
    )
    def k(table_hbm, idx_hbm, out_hbm, idx_v, rows_v, gsem):
        wid = lax.axis_index("s") * _NC + lax.axis_index("c")
        base = wid * _BPW
        pltpu.sync_copy(idx_hbm.at[pl.ds(base, _BPW)], idx_v)

        def start_gather(c, b):
            pltpu.async_copy(
                table_hbm.at[idx_v.at[pl.ds(c * _CHUNK, _CHUNK)]],
                rows_v.at[b],
                gsem,
            )

        def wait_gather(c, b):
            pltpu.make_async_copy(
                table_hbm.at[idx_v.at[pl.ds(c * _CHUNK, _CHUNK)]],
                rows_v.at[b],
                gsem,
            ).wait()

        def drain(c, b):
            wait_gather(c, b)
            pltpu.sync_copy(rows_v.at[b], out_hbm.at[pl.ds(base + c * _CHUNK, _CHUNK)])

        for b in range(_NBUF):
            start_gather(b, b)

        def group(gi, carry):
            for b in range(_NBUF):
                c = gi * _NBUF + b
                drain(c, b)
                start_gather(c + _NBUF, b)
            return carry

        lax.fori_loop(0, _NGROUPS - 1, group, 0)

        for b in range(_NBUF):
            drain((_NGROUPS - 1) * _NBUF + b, b)

    return k


_gather = _gather_fn()


def kernel(input_ids, table):
    ids = input_ids.reshape(-1).astype(jnp.int32)
    out = _gather(table, ids)
    return out.reshape(input_ids.shape + (D_E,))
